# baseline (speedup 1.0000x reference)
import jax
import jax.numpy as jnp
from jax.experimental import pallas as pl
from jax.experimental.pallas import tpu as pltpu

_SLABS = (32, 32)
_N = len(_SLABS)
_OFFS = tuple(sum(_SLABS[:i]) for i in range(_N))


def _copy_body(x_hbm, o_hbm, buf, lsem, ssem):
    def load(c):
        sl = pl.ds(_OFFS[c], _SLABS[c])
        return pltpu.make_async_copy(x_hbm.at[sl, :], buf.at[sl, :], lsem.at[c])

    def store(c):
        sl = pl.ds(_OFFS[c], _SLABS[c])
        return pltpu.make_async_copy(buf.at[sl, :], o_hbm.at[sl, :], ssem.at[c])

    for c in range(_N):
        load(c).start()
    for c in range(_N):
        load(c).wait()
        store(c).start()
    for c in range(_N):
        store(c).wait()


def kernel(logits, attention_mask):
    B, V = logits.shape
    out = pl.pallas_call(
        _copy_body,
        in_specs=[pl.BlockSpec(memory_space=pltpu.MemorySpace.HBM)],
        out_specs=pl.BlockSpec(memory_space=pltpu.MemorySpace.HBM),
        out_shape=jax.ShapeDtypeStruct((B, V), jnp.float32),
        scratch_shapes=[
            pltpu.VMEM((B, V), jnp.float32),
            pltpu.SemaphoreType.DMA((_N,)),
            pltpu.SemaphoreType.DMA((_N,)),
        ],
    )(logits)
    return out


# speedup vs baseline: 1.0154x; 1.0154x over previous
import jax
import jax.numpy as jnp
from jax.experimental import pallas as pl
from jax.experimental.pallas import tpu as pltpu

_SLABS = (16, 32, 16)
_N = len(_SLABS)
_OFFS = tuple(sum(_SLABS[:i]) for i in range(_N))


def _copy_body(x_hbm, o_hbm, buf, lsem, ssem):
    def load(c):
        sl = pl.ds(_OFFS[c], _SLABS[c])
        return pltpu.make_async_copy(x_hbm.at[sl, :], buf.at[sl, :], lsem.at[c])

    def store(c):
        sl = pl.ds(_OFFS[c], _SLABS[c])
        return pltpu.make_async_copy(buf.at[sl, :], o_hbm.at[sl, :], ssem.at[c])

    for c in range(_N):
        load(c).start()
    for c in range(_N):
        load(c).wait()
        store(c).start()
    for c in range(_N):
        store(c).wait()


def kernel(logits, attention_mask):
    B, V = logits.shape
    out = pl.pallas_call(
        _copy_body,
        in_specs=[pl.BlockSpec(memory_space=pltpu.MemorySpace.HBM)],
        out_specs=pl.BlockSpec(memory_space=pltpu.MemorySpace.HBM),
        out_shape=jax.ShapeDtypeStruct((B, V), jnp.float32),
        scratch_shapes=[
            pltpu.VMEM((B, V), jnp.float32),
            pltpu.SemaphoreType.DMA((_N,)),
            pltpu.SemaphoreType.DMA((_N,)),
        ],
    )(logits)
    return out
